# baseline (speedup 1.0000x reference)
import functools

import jax
import jax.numpy as jnp
from jax import lax
from jax.experimental import pallas as pl
from jax.experimental.pallas import tpu as pltpu
from jax.experimental.pallas import tpu_sc as plsc

MAXP = 100000
D = 64
NI = 4096
NJ = 200
NC, NS, L = 2, 16, 16
NW = NC * NS
IB = NI // NW
NHALF = 2 * (NJ // 2)


def _posenc_body(tT, table, out, idx_v, rows_v, tbuf, isems, gsems, osems):
    wid = lax.axis_index("s") * NC + lax.axis_index("c")
    i0 = wid * IB

    lane = lax.iota(jnp.int32, L)
    ridx = [lane + (cg * L) for cg in range(IB // L)]
    zero16 = jnp.zeros((L,), jnp.int32)

    def idx_copy(j, s):
        return pltpu.make_async_copy(
            tT.at[j, pl.ds(i0, IB)], idx_v.at[s], isems[s]
        )

    def gather_copy(s):
        return pltpu.make_async_copy(
            table.at[idx_v.at[s]], rows_v.at[s], gsems[s]
        )

    def out_copy(j, s):
        return pltpu.make_async_copy(
            tbuf.at[s], out.at[pl.ds(j * D, D), pl.ds(i0, IB)], osems[s]
        )

    def adjust(s):
        for i in range(IB // L):
            v = idx_v[s, pl.ds(i * L, L)]
            idx_v[s, pl.ds(i * L, L)] = jnp.where(v == 0, MAXP - 1, v - 1)

    def transpose(s):
        for d in range(D):
            cidx = zero16 + d
            for cg in range(IB // L):
                v = plsc.load_gather(rows_v.at[s], [ridx[cg], cidx])
                tbuf[s, d, pl.ds(cg * L, L)] = v

    def maybe(pred, fn):
        if pred is True:
            fn()
        else:
            pl.when(pred)(fn)

    def half(j, s, has_next, has_next2, has_prev2):
        gather_copy(s).wait()
        maybe(has_next2, lambda: idx_copy(j + 2, s).start())
        def _next():
            idx_copy(j + 1, 1 - s).wait()
            adjust(1 - s)
            gather_copy(1 - s).start()

        maybe(has_next, _next)
        maybe(has_prev2, lambda: out_copy(j - 2, s).wait())
        transpose(s)
        out_copy(j, s).start()

    def body(k, carry):
        not_last = k < NHALF // 2 - 1
        half(2 * k, 0, True, not_last, k > 0)
        half(2 * k + 1, 1, not_last, not_last, k > 0)
        return carry

    idx_copy(0, 0).start()
    idx_copy(0, 0).wait()
    adjust(0)
    gather_copy(0).start()
    idx_copy(1, 1).start()
    lax.fori_loop(0, NHALF // 2, body, 0)
    out_copy(NJ - 2, 0).wait()
    out_copy(NJ - 1, 1).wait()


_posenc_call = functools.partial(
    pl.kernel,
    mesh=plsc.VectorSubcoreMesh(core_axis_name="c", subcore_axis_name="s"),
    out_type=jax.ShapeDtypeStruct((NJ * D, NI), jnp.float32),
    scratch_types=[
        pltpu.VMEM((2, IB), jnp.int32),
        pltpu.VMEM((2, IB, D), jnp.float32),
        pltpu.VMEM((2, D, IB), jnp.float32),
        [pltpu.SemaphoreType.DMA] * 2,
        [pltpu.SemaphoreType.DMA] * 2,
        [pltpu.SemaphoreType.DMA] * 2,
    ],
    compiler_params=pltpu.CompilerParams(
        use_tc_tiling_on_sc=False, needs_layout_passes=False
    ),
)(_posenc_body)


@jax.jit
def kernel(t, pos_enc):
    tT = t.T
    out = _posenc_call(tT, pos_enc)
    return out.T.reshape(NI, NJ, D)


# speedup vs baseline: 4.1756x; 4.1756x over previous
import functools

import jax
import jax.numpy as jnp
from jax import lax
from jax.experimental import pallas as pl
from jax.experimental.pallas import tpu as pltpu
from jax.experimental.pallas import tpu_sc as plsc

MAXP = 100000
D = 64
DP = 128
B_TOTAL = 4096 * 200
NC, NS, L = 2, 16, 16
NW = NC * NS
B_PER_W = B_TOTAL // NW
IDXW = 128
NSTREAM = 4
HALF = NSTREAM * IDXW
CHUNK = 2 * HALF
NCHUNK = B_PER_W // CHUNK
IROWS_PER_W = B_PER_W // IDXW


def _posenc_body(t2, table, out, idx_v, rows_v, isem, gsems, osems):
    wid = lax.axis_index("s") * NC + lax.axis_index("c")
    base = wid * B_PER_W
    irow0 = wid * IROWS_PER_W

    def idx_copy(ci):
        return pltpu.make_async_copy(
            t2.at[pl.ds(irow0 + ci * 2 * NSTREAM, 2 * NSTREAM)], idx_v, isem
        )

    def out_copy(ci, h):
        return pltpu.make_async_copy(
            rows_v.at[h],
            out.at[pl.ds(base + ci * CHUNK + h * HALF, HALF), pl.ds(0, D)],
            osems[h],
        )

    def fire_gathers(h):
        return [
            pltpu.async_copy(
                table.at[idx_v.at[h * NSTREAM + j]],
                rows_v.at[h, pl.ds(j * IDXW, IDXW)],
                gsems[h],
            )
            for j in range(NSTREAM)
        ]

    def body(ci, carry):
        idx_copy(ci).wait()
        for j in range(2 * NSTREAM):
            for i in range(IDXW // L):
                v = idx_v[j, pl.ds(i * L, L)]
                idx_v[j, pl.ds(i * L, L)] = jnp.where(v == 0, MAXP - 1, v - 1)

        @pl.when(ci > 0)
        def _():
            out_copy(ci - 1, 0).wait()

        h0 = fire_gathers(0)

        @pl.when(ci > 0)
        def _():
            out_copy(ci - 1, 1).wait()

        h1 = fire_gathers(1)
        for hd in h0:
            hd.wait()
        out_copy(ci, 0).start()
        for hd in h1:
            hd.wait()

        @pl.when(ci < NCHUNK - 1)
        def _():
            idx_copy(ci + 1).start()

        out_copy(ci, 1).start()
        return carry

    idx_copy(0).start()
    lax.fori_loop(0, NCHUNK, body, 0)
    out_copy(NCHUNK - 1, 0).wait()
    out_copy(NCHUNK - 1, 1).wait()


_posenc_call = functools.partial(
    pl.kernel,
    mesh=plsc.VectorSubcoreMesh(core_axis_name="c", subcore_axis_name="s"),
    out_type=jax.ShapeDtypeStruct((B_TOTAL, DP), jnp.float32),
    scratch_types=[
        pltpu.VMEM((2 * NSTREAM, IDXW), jnp.int32),
        pltpu.VMEM((2, HALF, D), jnp.float32),
        pltpu.SemaphoreType.DMA,
        [pltpu.SemaphoreType.DMA] * 2,
        [pltpu.SemaphoreType.DMA] * 2,
    ],
    compiler_params=pltpu.CompilerParams(use_tc_tiling_on_sc=False),
)(_posenc_body)


@jax.jit
def kernel(t, pos_enc):
    t2 = t.reshape(B_TOTAL // IDXW, IDXW)
    out = _posenc_call(t2, pos_enc)
    return out[:, :D].reshape(t.shape + (D,))


# speedup vs baseline: 4.2275x; 1.0124x over previous
import functools

import jax
import jax.numpy as jnp
from jax import lax
from jax.experimental import pallas as pl
from jax.experimental.pallas import tpu as pltpu
from jax.experimental.pallas import tpu_sc as plsc

MAXP = 100000
D = 64
DP = 128
B_TOTAL = 4096 * 200
NC, NS, L = 2, 16, 16
NW = NC * NS
B_PER_W = B_TOTAL // NW
IDXW = 128
NSLOT = 4
NSTREAM = 2
PART = NSTREAM * IDXW
CHUNK = NSLOT * PART
NCHUNK = B_PER_W // CHUNK
IROWS_PER_W = B_PER_W // IDXW


def _posenc_body(t2, table, out, idx_v, rows_v, isem, gsems, osems):
    wid = lax.axis_index("s") * NC + lax.axis_index("c")
    base = wid * B_PER_W
    irow0 = wid * IROWS_PER_W

    def idx_copy(ci):
        return pltpu.make_async_copy(
            t2.at[pl.ds(irow0 + ci * NSLOT * NSTREAM, NSLOT * NSTREAM)],
            idx_v,
            isem,
        )

    def out_copy(ci, h):
        return pltpu.make_async_copy(
            rows_v.at[h],
            out.at[pl.ds(base + ci * CHUNK + h * PART, PART), pl.ds(0, D)],
            osems[h],
        )

    def fire_gathers(h):
        return [
            pltpu.async_copy(
                table.at[idx_v.at[h * NSTREAM + j]],
                rows_v.at[h, pl.ds(j * IDXW, IDXW)],
                gsems[h],
            )
            for j in range(NSTREAM)
        ]

    def body(ci, carry):
        idx_copy(ci).wait()
        for j in range(NSLOT * NSTREAM):
            for i in range(IDXW // L):
                v = idx_v[j, pl.ds(i * L, L)]
                idx_v[j, pl.ds(i * L, L)] = jnp.where(v == 0, MAXP - 1, v - 1)

        handles = []
        for h in range(NSLOT):
            @pl.when(ci > 0)
            def _(h=h):
                out_copy(ci - 1, h).wait()

            handles.append(fire_gathers(h))

        for h in range(NSLOT):
            for hd in handles[h]:
                hd.wait()
            if h == NSLOT - 1:
                @pl.when(ci < NCHUNK - 1)
                def _():
                    idx_copy(ci + 1).start()

            out_copy(ci, h).start()
        return carry

    idx_copy(0).start()
    lax.fori_loop(0, NCHUNK, body, 0)
    for h in range(NSLOT):
        out_copy(NCHUNK - 1, h).wait()


_posenc_call = functools.partial(
    pl.kernel,
    mesh=plsc.VectorSubcoreMesh(core_axis_name="c", subcore_axis_name="s"),
    out_type=jax.ShapeDtypeStruct((B_TOTAL, DP), jnp.float32),
    scratch_types=[
        pltpu.VMEM((NSLOT * NSTREAM, IDXW), jnp.int32),
        pltpu.VMEM((NSLOT, PART, D), jnp.float32),
        pltpu.SemaphoreType.DMA,
        [pltpu.SemaphoreType.DMA] * NSLOT,
        [pltpu.SemaphoreType.DMA] * NSLOT,
    ],
    compiler_params=pltpu.CompilerParams(use_tc_tiling_on_sc=False),
)(_posenc_body)


@jax.jit
def kernel(t, pos_enc):
    t2 = t.reshape(B_TOTAL // IDXW, IDXW)
    out = _posenc_call(t2, pos_enc)
    return out[:, :D].reshape(t.shape + (D,))
